# baseline (speedup 1.0000x reference)
import functools

import jax
import jax.numpy as jnp
from jax import lax
from jax.experimental import pallas as pl
from jax.experimental.pallas import tpu as pltpu
from jax.experimental.pallas import tpu_sc as plsc

N = 10000
E = 320000
C = 128
NC = 2
NS = 16
NW = NC * NS
EPW = E // NW
CH = 80
NCHUNK = EPW // CH
NRING = 3
DW = 16
WCH = 80
AUXZ = 104
NWB = N // WCH
ITS = (NWB + NS - 1) // NS

_SC_PARAMS = pltpu.CompilerParams(use_tc_tiling_on_sc=False)
_MESH = dict(core_axis_name="c", subcore_axis_name="s")


def _sc_aggregate(x, src_r, dst_r, zeros):
    @functools.partial(
        pl.kernel,
        out_type=jax.ShapeDtypeStruct((NC, N, C), jnp.float32),
        mesh=plsc.VectorSubcoreMesh(**_MESH),
        compiler_params=_SC_PARAMS,
        scratch_types=[
            pltpu.VMEM((NCHUNK, CH), jnp.int32),
            pltpu.VMEM((NCHUNK, CH), jnp.int32),
            pltpu.VMEM((NRING, CH, C), jnp.float32),
            pltpu.VMEM_SHARED((N, C), jnp.float32),
        ] + [pltpu.SemaphoreType.DMA] * (1 + NRING),
    )
    def agg(x_hbm, src_hbm, dst_hbm, z_hbm, out_hbm,
            src_v, dst_v, rows_v, acc_sh, gsem, *ssem):
        cid = lax.axis_index("c")
        sid = lax.axis_index("s")
        wid = cid * NS + sid

        pltpu.sync_copy(z_hbm, rows_v.at[0])

        for it in range(ITS):
            cidx = it * NS + sid

            @pl.when(cidx < NWB)
            def _zero_chunk():
                pltpu.sync_copy(rows_v.at[0],
                                acc_sh.at[pl.ds(cidx * WCH, WCH)])

        pltpu.sync_copy(src_hbm.at[wid], src_v)
        pltpu.sync_copy(dst_hbm.at[wid], dst_v)
        plsc.subcore_barrier()

        def fire_g(j, b):
            pltpu.async_copy(x_hbm.at[src_v.at[j]], rows_v.at[b], gsem)

        def drain_g(j, b):
            pltpu.make_async_copy(
                x_hbm.at[src_v.at[j]], rows_v.at[b], gsem).wait()

        def fire_s(j, b):
            pltpu.async_copy(rows_v.at[b], acc_sh.at[dst_v.at[j]],
                             ssem[b], add=True)

        def wait_s(j, b):
            pltpu.make_async_copy(rows_v.at[b], acc_sh.at[dst_v.at[j]],
                                  ssem[b]).wait()

        fire_g(0, 0)

        @pl.loop(0, NCHUNK // NRING)
        def _grp(h):
            for s in range(NRING):
                j = NRING * h + s
                b2 = (s + 1) % NRING
                drain_g(j, s)
                fire_s(j, s)

                @pl.when(j >= 2)
                def _wait_prev():
                    wait_s(j - 2, b2)

                @pl.when(j + 1 < NCHUNK)
                def _prefetch():
                    fire_g(j + 1, b2)

        for j in range(NRING * (NCHUNK // NRING), NCHUNK):
            s = j % NRING
            b2 = (s + 1) % NRING
            drain_g(j, s)
            fire_s(j, s)
            wait_s(j - 2, b2)
            if j + 1 < NCHUNK:
                fire_g(j + 1, b2)
        wait_s(NCHUNK - 2, (NCHUNK - 2) % NRING)
        wait_s(NCHUNK - 1, (NCHUNK - 1) % NRING)

        plsc.subcore_barrier()

        for it in range(ITS):
            cidx = it * NS + sid

            @pl.when(cidx < NWB)
            def _write_chunk():
                r0 = cidx * WCH
                pltpu.sync_copy(acc_sh.at[pl.ds(r0, WCH)], rows_v.at[0])
                pltpu.sync_copy(rows_v.at[0],
                                out_hbm.at[cid, pl.ds(r0, WCH)])

    return agg(x, src_r, dst_r, zeros)


def _sc_degree(dst_r, aux):
    @functools.partial(
        pl.kernel,
        out_type=jax.ShapeDtypeStruct((NC, N, DW), jnp.float32),
        mesh=plsc.VectorSubcoreMesh(**_MESH),
        compiler_params=_SC_PARAMS,
        scratch_types=[
            pltpu.VMEM((NCHUNK, CH), jnp.int32),
            pltpu.VMEM((CH, DW), jnp.float32),
            pltpu.VMEM((WCH, DW), jnp.float32),
            pltpu.VMEM_SHARED((N, DW), jnp.float32),
        ],
    )
    def deg(dst_hbm, aux_hbm, deg_out,
            dst_v, ones_v, dbuf, deg_sh):
        cid = lax.axis_index("c")
        sid = lax.axis_index("s")
        wid = cid * NS + sid

        pltpu.sync_copy(aux_hbm.at[pl.ds(0, CH)], ones_v)
        pltpu.sync_copy(aux_hbm.at[pl.ds(AUXZ, WCH)], dbuf)

        for it in range(ITS):
            cidx = it * NS + sid

            @pl.when(cidx < NWB)
            def _zero_chunk():
                pltpu.sync_copy(dbuf, deg_sh.at[pl.ds(cidx * WCH, WCH)])

        pltpu.sync_copy(dst_hbm.at[wid], dst_v)
        plsc.subcore_barrier()

        @pl.loop(0, NCHUNK)
        def _edges(j):
            pltpu.sync_copy(ones_v, deg_sh.at[dst_v.at[j]], add=True)

        plsc.subcore_barrier()

        for it in range(ITS):
            cidx = it * NS + sid

            @pl.when(cidx < NWB)
            def _write_chunk():
                r0 = cidx * WCH
                pltpu.sync_copy(deg_sh.at[pl.ds(r0, WCH)], dbuf)
                pltpu.sync_copy(dbuf, deg_out.at[cid, pl.ds(r0, WCH)])

    return deg(dst_r, aux)


BR = 2000


def _tc_body(p_ref, deg_ref, x_ref, wlT_ref, wrT_ref, b_ref, o_ref):
    s = p_ref[0] + p_ref[1]
    d = deg_ref[0, :, 0:1] + deg_ref[1, :, 0:1]
    m = s / jnp.maximum(d, 1.0)
    o_ref[...] = (
        jnp.dot(m, wlT_ref[...], preferred_element_type=jnp.float32,
                precision=lax.Precision.HIGHEST)
        + jnp.dot(x_ref[...], wrT_ref[...], preferred_element_type=jnp.float32,
                  precision=lax.Precision.HIGHEST)
        + b_ref[...]
    )


def _tc_combine(p, degp, x, wlT, wrT, b):
    return pl.pallas_call(
        _tc_body,
        grid=(N // BR,),
        in_specs=[
            pl.BlockSpec((NC, BR, C), lambda i: (0, i, 0)),
            pl.BlockSpec((NC, BR, DW), lambda i: (0, i, 0)),
            pl.BlockSpec((BR, C), lambda i: (i, 0)),
            pl.BlockSpec((C, C), lambda i: (0, 0)),
            pl.BlockSpec((C, C), lambda i: (0, 0)),
            pl.BlockSpec((1, C), lambda i: (0, 0)),
        ],
        out_specs=pl.BlockSpec((BR, C), lambda i: (i, 0)),
        out_shape=jax.ShapeDtypeStruct((N, C), jnp.float32),
    )(p, degp, x, wlT, wrT, b)


def kernel(x, edge_index, W_l, b_l, W_r):
    src = edge_index[0].reshape(NW, NCHUNK, CH)
    dst = edge_index[1].reshape(NW, NCHUNK, CH)
    zeros = jnp.zeros((CH, C), jnp.float32)
    aux = jnp.concatenate(
        [jnp.ones((CH, DW), jnp.float32),
         jnp.zeros((AUXZ - CH + WCH, DW), jnp.float32)])
    parts = _sc_aggregate(x, src, dst, zeros)
    degs = _sc_degree(dst, aux)
    return _tc_combine(parts, degs, x, W_l.T, W_r.T, b_l.reshape(1, C))


# speedup vs baseline: 1.1208x; 1.1208x over previous
import functools

import jax
import jax.numpy as jnp
from jax import lax
from jax.experimental import pallas as pl
from jax.experimental.pallas import tpu as pltpu
from jax.experimental.pallas import tpu_sc as plsc

N = 10000
E = 320000
C = 128
NC = 2
NS = 16
NW = NC * NS
EPW = E // NW
CH = 125
NCHUNK = EPW // CH
NHALF = 2
CPH = NCHUNK // NHALF
DW = 16
WCH = 80
NWB = N // WCH
ITS = (NWB + NS - 1) // NS
AUXZ = 128

_SC_PARAMS = pltpu.CompilerParams(use_tc_tiling_on_sc=False)
_MESH = dict(core_axis_name="c", subcore_axis_name="s")


def _sc_aggregate(x, src_r, dst_r, zeros):
    @functools.partial(
        pl.kernel,
        out_type=jax.ShapeDtypeStruct((NC, N, C), jnp.float32),
        mesh=plsc.VectorSubcoreMesh(**_MESH),
        compiler_params=_SC_PARAMS,
        scratch_types=[
            pltpu.VMEM((CPH, CH), jnp.int32),
            pltpu.VMEM((CPH, CH), jnp.int32),
            pltpu.VMEM((2, CH, C), jnp.float32),
            pltpu.VMEM_SHARED((N, C), jnp.float32),
            pltpu.SemaphoreType.DMA,
        ],
    )
    def agg(x_hbm, src_hbm, dst_hbm, z_hbm, out_hbm,
            src_v, dst_v, rows_v, acc_sh, sem):
        cid = lax.axis_index("c")
        sid = lax.axis_index("s")
        wid = cid * NS + sid

        pltpu.sync_copy(z_hbm, rows_v.at[0])

        for it in range(ITS):
            cidx = it * NS + sid

            @pl.when(cidx < NWB)
            def _zero_chunk():
                pltpu.sync_copy(rows_v.at[0, pl.ds(0, WCH)],
                                acc_sh.at[pl.ds(cidx * WCH, WCH)])

        plsc.subcore_barrier()

        def fire(j, pb):
            pltpu.async_copy(x_hbm.at[src_v.at[j]], rows_v.at[pb], sem)

        def drain(j, pb):
            pltpu.make_async_copy(
                x_hbm.at[src_v.at[j]], rows_v.at[pb], sem).wait()

        def scatter(j, pb):
            pltpu.sync_copy(rows_v.at[pb],
                            acc_sh.at[dst_v.at[j]], add=True)

        for half in range(NHALF):
            pltpu.sync_copy(src_hbm.at[wid, half], src_v)
            pltpu.sync_copy(dst_hbm.at[wid, half], dst_v)
            fire(0, 0)

            @pl.loop(0, CPH // 2)
            def _grp(h):
                j0 = 2 * h
                drain(j0, 0)
                fire(j0 + 1, 1)
                scatter(j0, 0)
                j1 = j0 + 1
                drain(j1, 1)

                @pl.when(j1 + 1 < CPH)
                def _refill():
                    fire(j1 + 1, 0)

                scatter(j1, 1)

        plsc.subcore_barrier()

        for it in range(ITS):
            cidx = it * NS + sid

            @pl.when(cidx < NWB)
            def _write_chunk():
                r0 = cidx * WCH
                pltpu.sync_copy(acc_sh.at[pl.ds(r0, WCH)],
                                rows_v.at[0, pl.ds(0, WCH)])
                pltpu.sync_copy(rows_v.at[0, pl.ds(0, WCH)],
                                out_hbm.at[cid, pl.ds(r0, WCH)])

    return agg(x, src_r, dst_r, zeros)


def _sc_degree(dst_r, aux):
    @functools.partial(
        pl.kernel,
        out_type=jax.ShapeDtypeStruct((NC, N, DW), jnp.float32),
        mesh=plsc.VectorSubcoreMesh(**_MESH),
        compiler_params=_SC_PARAMS,
        scratch_types=[
            pltpu.VMEM((NHALF, CPH, CH), jnp.int32),
            pltpu.VMEM((CH, DW), jnp.float32),
            pltpu.VMEM((WCH, DW), jnp.float32),
            pltpu.VMEM_SHARED((N, DW), jnp.float32),
        ],
    )
    def deg(dst_hbm, aux_hbm, deg_out,
            dst_v, ones_v, dbuf, deg_sh):
        cid = lax.axis_index("c")
        sid = lax.axis_index("s")
        wid = cid * NS + sid

        pltpu.sync_copy(aux_hbm.at[pl.ds(0, CH)], ones_v)
        pltpu.sync_copy(aux_hbm.at[pl.ds(AUXZ, WCH)], dbuf)

        for it in range(ITS):
            cidx = it * NS + sid

            @pl.when(cidx < NWB)
            def _zero_chunk():
                pltpu.sync_copy(dbuf, deg_sh.at[pl.ds(cidx * WCH, WCH)])

        pltpu.sync_copy(dst_hbm.at[wid], dst_v)
        plsc.subcore_barrier()

        for half in range(NHALF):

            @pl.loop(0, CPH)
            def _edges(j):
                pltpu.sync_copy(ones_v, deg_sh.at[dst_v.at[half, j]], add=True)

        plsc.subcore_barrier()

        for it in range(ITS):
            cidx = it * NS + sid

            @pl.when(cidx < NWB)
            def _write_chunk():
                r0 = cidx * WCH
                pltpu.sync_copy(deg_sh.at[pl.ds(r0, WCH)], dbuf)
                pltpu.sync_copy(dbuf, deg_out.at[cid, pl.ds(r0, WCH)])

    return deg(dst_r, aux)


BR = 2000


def _tc_body(p_ref, deg_ref, x_ref, wlT_ref, wrT_ref, b_ref, o_ref):
    s = p_ref[0] + p_ref[1]
    d = deg_ref[0, :, 0:1] + deg_ref[1, :, 0:1]
    m = s / jnp.maximum(d, 1.0)
    o_ref[...] = (
        jnp.dot(m, wlT_ref[...], preferred_element_type=jnp.float32,
                precision=lax.Precision.HIGHEST)
        + jnp.dot(x_ref[...], wrT_ref[...], preferred_element_type=jnp.float32,
                  precision=lax.Precision.HIGHEST)
        + b_ref[...]
    )


def _tc_combine(p, degp, x, wlT, wrT, b):
    return pl.pallas_call(
        _tc_body,
        grid=(N // BR,),
        in_specs=[
            pl.BlockSpec((NC, BR, C), lambda i: (0, i, 0)),
            pl.BlockSpec((NC, BR, DW), lambda i: (0, i, 0)),
            pl.BlockSpec((BR, C), lambda i: (i, 0)),
            pl.BlockSpec((C, C), lambda i: (0, 0)),
            pl.BlockSpec((C, C), lambda i: (0, 0)),
            pl.BlockSpec((1, C), lambda i: (0, 0)),
        ],
        out_specs=pl.BlockSpec((BR, C), lambda i: (i, 0)),
        out_shape=jax.ShapeDtypeStruct((N, C), jnp.float32),
    )(p, degp, x, wlT, wrT, b)


def kernel(x, edge_index, W_l, b_l, W_r):
    src = edge_index[0].reshape(NW, NHALF, CPH, CH)
    dst = edge_index[1].reshape(NW, NHALF, CPH, CH)
    zeros = jnp.zeros((CH, C), jnp.float32)
    aux = jnp.concatenate(
        [jnp.ones((CH, DW), jnp.float32),
         jnp.zeros((AUXZ - CH + WCH, DW), jnp.float32)])
    parts = _sc_aggregate(x, src, dst, zeros)
    degs = _sc_degree(dst, aux)
    return _tc_combine(parts, degs, x, W_l.T, W_r.T, b_l.reshape(1, C))
